# baseline (speedup 1.0000x reference)
import functools

import jax
import jax.numpy as jnp
from jax import lax
from jax.experimental import pallas as pl
from jax.experimental.pallas import tpu as pltpu, tpu_sc as plsc

_NUM_TASKS = 1000
_EMBED_DIM = 128
_BATCH = 4096

_SC_BATCH = 2048
_TC_BATCH = _BATCH - _SC_BATCH
_TC_BLOCK = 256
_K_PAD = 1024

_info = plsc.get_sparse_core_info()
_NC, _NS = _info.num_cores, _info.num_subcores
_NW = _NC * _NS
_B_PER_W = _SC_BATCH // _NW


def _make_sc_gather():
    mesh = plsc.VectorSubcoreMesh(core_axis_name="c", subcore_axis_name="s")

    @functools.partial(
        pl.kernel,
        mesh=mesh,
        out_type=jax.ShapeDtypeStruct((_SC_BATCH, _EMBED_DIM), jnp.float32),
        scratch_types=[
            pltpu.VMEM((_B_PER_W,), jnp.int32),
            pltpu.VMEM((_B_PER_W, _EMBED_DIM), jnp.float32),
            pltpu.SemaphoreType.DMA,
        ],
    )
    def gather(idx_hbm, table_hbm, out_hbm, idx_v, rows_v, sem):
        wid = lax.axis_index("s") * _NC + lax.axis_index("c")
        base = wid * _B_PER_W
        pltpu.sync_copy(idx_hbm.at[pl.ds(base, _B_PER_W)], idx_v)
        pltpu.async_copy(table_hbm.at[idx_v], rows_v, sem).wait()
        pltpu.sync_copy(rows_v, out_hbm.at[pl.ds(base, _B_PER_W)])

    return gather


_sc_gather = _make_sc_gather()


def _tc_body(tid_ref, w_ref, out_ref):
    tid = tid_ref[:, :]
    cols = lax.broadcasted_iota(jnp.int32, (_TC_BLOCK, _K_PAD), 1)
    onehot = (tid == cols).astype(jnp.float32)
    out_ref[:, :] = jnp.dot(
        onehot, w_ref[:, :], preferred_element_type=jnp.float32
    )


def _tc_gather(tid2d, w_pad):
    return pl.pallas_call(
        _tc_body,
        grid=(_TC_BATCH // _TC_BLOCK,),
        in_specs=[
            pl.BlockSpec((_TC_BLOCK, 1), lambda i: (i, 0)),
            pl.BlockSpec((_K_PAD, _EMBED_DIM), lambda i: (0, 0)),
        ],
        out_specs=pl.BlockSpec((_TC_BLOCK, _EMBED_DIM), lambda i: (i, 0)),
        out_shape=jax.ShapeDtypeStruct((_TC_BATCH, _EMBED_DIM), jnp.float32),
    )(tid2d, w_pad)


def kernel(batch_size, task_id, weight):
    tid = task_id.astype(jnp.int32)
    rows_sc = _sc_gather(tid[:_SC_BATCH], weight)
    w_pad = jnp.pad(weight, ((0, _K_PAD - _NUM_TASKS), (0, 0)))
    rows_tc = _tc_gather(tid[_SC_BATCH:, None], w_pad)
    rows = jnp.concatenate([rows_sc, rows_tc], axis=0)
    return rows[:, None, :]


# speedup vs baseline: 1.3540x; 1.3540x over previous
import functools

import jax
import jax.numpy as jnp
from jax import lax
from jax.experimental import pallas as pl
from jax.experimental.pallas import tpu as pltpu, tpu_sc as plsc

_NUM_TASKS = 1000
_EMBED_DIM = 128
_BATCH = 4096

_info = plsc.get_sparse_core_info()
_NC, _NS = _info.num_cores, _info.num_subcores
_NW = _NC * _NS
_B_PER_W = _BATCH // _NW


def _make_gather():
    mesh = plsc.VectorSubcoreMesh(core_axis_name="c", subcore_axis_name="s")

    @functools.partial(
        pl.kernel,
        mesh=mesh,
        out_type=jax.ShapeDtypeStruct((_BATCH, _EMBED_DIM), jnp.float32),
        scratch_types=[
            pltpu.VMEM((_B_PER_W,), jnp.int32),
            pltpu.VMEM((_B_PER_W, _EMBED_DIM), jnp.float32),
            pltpu.SemaphoreType.DMA,
        ],
    )
    def gather(idx_hbm, table_hbm, out_hbm, idx_v, rows_v, sem):
        wid = lax.axis_index("s") * _NC + lax.axis_index("c")
        base = wid * _B_PER_W
        pltpu.sync_copy(idx_hbm.at[pl.ds(base, _B_PER_W)], idx_v)
        pltpu.async_copy(table_hbm.at[idx_v], rows_v, sem).wait()
        pltpu.sync_copy(rows_v, out_hbm.at[pl.ds(base, _B_PER_W)])

    return gather


_gather = _make_gather()


def kernel(batch_size, task_id, weight):
    rows = _gather(task_id.astype(jnp.int32), weight)
    return rows[:, None, :]
